# baseline (device time: 24929 ns/iter reference)
import jax
import jax.numpy as jnp
from jax import lax
from jax.experimental import pallas as pl
from jax.experimental.pallas import tpu as pltpu

N_SUB = 4
BLK = 64


def kernel(x, Wq, K_ext, V_ext, Wo):
    B, Sq, Dm = x.shape
    _, Skv, Hq, Dh = K_ext.shape
    TRIPLES = [(b, q, h) for b in range(B) for q in range(2) for h in range(Hq)]

    def body(x_ref, wq_ref, k_ref, v_ref, wo_ref, out_ref,
             kbuf, vbuf, kt_send, vt_send, send_sems, recv_sems):
        my = lax.axis_index("i")
        parity = lax.rem(my, 2)
        my_t = my // 2

        barrier = pltpu.get_barrier_semaphore()
        for u in range(N_SUB):
            @pl.when(my_t != u)
            def _():
                pl.semaphore_signal(
                    barrier, inc=1,
                    device_id=(parity + 2 * u,),
                    device_id_type=pl.DeviceIdType.MESH,
                )
        pl.semaphore_wait(barrier, N_SUB - 1)

        kt_val = jnp.transpose(k_ref[...], (0, 2, 1, 3)).astype(jnp.bfloat16)
        kt_send[...] = kt_val

        def sends(src, buf, c):
            for t in range(N_SUB):
                @pl.when(my_t == t)
                def _():
                    for u in (t ^ 2, t ^ 1, t ^ 3):
                        pltpu.make_async_remote_copy(
                            src_ref=src,
                            dst_ref=buf.at[t],
                            send_sem=send_sems.at[u, c],
                            recv_sem=recv_sems.at[t, c],
                            device_id=(parity + 2 * u,),
                            device_id_type=pl.DeviceIdType.MESH,
                        ).start()

        sends(kt_send, kbuf, 0)
        vt_val = jnp.transpose(v_ref[...], (0, 2, 1, 3)).astype(jnp.bfloat16)
        vt_send[...] = vt_val
        sends(vt_send, vbuf, 1)

        qs = [
            jnp.dot(x_ref[b], wq_ref[...], preferred_element_type=jnp.float32)
            for b in range(B)
        ]
        q2 = {
            (b, q, h): qs[b][q * BLK:(q + 1) * BLK, h * Dh:(h + 1) * Dh]
            .astype(jnp.bfloat16)
            for (b, q, h) in TRIPLES
        }

        def partial(bqh, kv, vv):
            s = lax.dot_general(
                q2[bqh], kv, (((1,), (1,)), ((), ())),
                preferred_element_type=jnp.float32,
            ) * 0.125
            m = jnp.max(s, axis=-1, keepdims=True)
            p = jnp.exp(s - m)
            l = jnp.sum(p, axis=-1, keepdims=True)
            acc = jnp.dot(p.astype(jnp.bfloat16), vv,
                          preferred_element_type=jnp.float32)
            return m, l, acc

        pA = {
            bqh: partial(
                bqh,
                kt_val[bqh[0], bqh[2], bqh[1] * BLK:(bqh[1] + 1) * BLK, :],
                vt_val[bqh[0], bqh[2], bqh[1] * BLK:(bqh[1] + 1) * BLK, :],
            )
            for bqh in TRIPLES
        }

        for t in range(N_SUB):
            @pl.when(my_t == t)
            def _():
                near, mid, far = t ^ 2, t ^ 1, t ^ 3

                def wait_slot(slot):
                    for c, buf in enumerate((kbuf, vbuf)):
                        pltpu.make_async_remote_copy(
                            src_ref=kt_send,
                            dst_ref=buf.at[slot],
                            send_sem=send_sems.at[0, c],
                            recv_sem=recv_sems.at[slot, c],
                            device_id=(0,),
                            device_id_type=pl.DeviceIdType.MESH,
                        ).wait_recv()

                wait_slot(near)
                wait_slot(mid)
                kn, km = kbuf[near], kbuf[mid]
                vn, vm = vbuf[near], vbuf[mid]
                pB = {}
                for bqh in TRIPLES:
                    b, q, h = bqh
                    kv = jnp.concatenate(
                        [kn[b, h, q * BLK:(q + 1) * BLK, :],
                         km[b, h, q * BLK:(q + 1) * BLK, :]], axis=0)
                    vv = jnp.concatenate(
                        [vn[b, h, q * BLK:(q + 1) * BLK, :],
                         vm[b, h, q * BLK:(q + 1) * BLK, :]], axis=0)
                    pB[bqh] = partial(bqh, kv, vv)

                wait_slot(far)
                kf, vf = kbuf[far], vbuf[far]
                row_blocks = []
                for b in range(B):
                    for q in range(2):
                        head_blocks = []
                        for h in range(Hq):
                            bqh = (b, q, h)
                            mC, lC, aC = partial(
                                bqh,
                                kf[b, h, q * BLK:(q + 1) * BLK, :],
                                vf[b, h, q * BLK:(q + 1) * BLK, :],
                            )
                            mA, lA, aA = pA[bqh]
                            mB, lB, aB = pB[bqh]
                            M = jnp.maximum(jnp.maximum(mA, mB), mC)
                            wA = jnp.exp(mA - M)
                            wB = jnp.exp(mB - M)
                            wC = jnp.exp(mC - M)
                            l = lA * wA + lB * wB + lC * wC
                            ctx = (aA * wA + aB * wB + aC * wC) / l
                            head_blocks.append(ctx)
                        row_blocks.append(jnp.concatenate(head_blocks, axis=1))
                cm = jnp.concatenate(row_blocks, axis=0)
                om = jnp.dot(cm, wo_ref[...],
                             preferred_element_type=jnp.float32)
                for b in range(B):
                    out_ref[b] = om[b * Sq:(b + 1) * Sq, :]

                for u in (near, mid, far):
                    for c, buf in enumerate((kbuf, vbuf)):
                        pltpu.make_async_remote_copy(
                            src_ref=kt_send,
                            dst_ref=buf.at[t],
                            send_sem=send_sems.at[u, c],
                            recv_sem=recv_sems.at[t, c],
                            device_id=(0,),
                            device_id_type=pl.DeviceIdType.MESH,
                        ).wait_send()

    return pl.pallas_call(
        body,
        out_shape=jax.ShapeDtypeStruct((B, Sq, Dm), jnp.float32),
        in_specs=[pl.BlockSpec(memory_space=pltpu.VMEM)] * 5,
        out_specs=pl.BlockSpec(memory_space=pltpu.VMEM),
        scratch_shapes=[
            pltpu.VMEM((N_SUB, B, Hq, Skv, Dh), jnp.bfloat16),
            pltpu.VMEM((N_SUB, B, Hq, Skv, Dh), jnp.bfloat16),
            pltpu.VMEM((B, Hq, Skv, Dh), jnp.bfloat16),
            pltpu.VMEM((B, Hq, Skv, Dh), jnp.bfloat16),
            pltpu.SemaphoreType.DMA((N_SUB, 2)),
            pltpu.SemaphoreType.DMA((N_SUB, 2)),
        ],
        compiler_params=pltpu.CompilerParams(collective_id=0),
    )(x, Wq, K_ext, V_ext, Wo)
